# baseline (speedup 1.0000x reference)
import functools

import jax
import jax.numpy as jnp
from jax import lax
from jax.experimental import pallas as pl
from jax.experimental.pallas import tpu as pltpu
from jax.experimental.pallas import tpu_sc as plsc

VOCAB = 1000000
DIM = 32
NVALS = 81920
CDIM = 4 * DIM

_info = plsc.get_sparse_core_info()
_NC, _NS = _info.num_cores, _info.num_subcores
_NW = _NC * _NS
_BPW = NVALS // _NW
_CHUNK = 256
_NCHUNK = _BPW // _CHUNK
_L = 16


_mesh = plsc.VectorSubcoreMesh(core_axis_name="c", subcore_axis_name="s")


@functools.partial(
    pl.kernel,
    mesh=_mesh,
    out_type=[jax.ShapeDtypeStruct((DIM, NVALS), jnp.float32)] * 4,
    scratch_types=[
        pltpu.VMEM((_BPW,), jnp.int32),
        pltpu.VMEM((_CHUNK, CDIM), jnp.float32),
        pltpu.VMEM((_CHUNK, CDIM), jnp.float32),
        pltpu.VMEM((DIM, _CHUNK), jnp.float32),
        pltpu.SemaphoreType.DMA,
    ],
    compiler_params=pltpu.CompilerParams(needs_layout_passes=False),
)
def _gather4(v1, v2, v3, v4, tab, o1, o2, o3, o4,
             idx_v, rows_a, rows_b, tr_v, sem):
    wid = lax.axis_index("s") * _NC + lax.axis_index("c")
    base = wid * _BPW
    bufs = (rows_a, rows_b)
    lane = lax.iota(jnp.int32, _L)
    for f, (vals, out) in enumerate(((v1, o1), (v2, o2), (v3, o3), (v4, o4))):
        pltpu.sync_copy(vals.at[pl.ds(base, _BPW)], idx_v)
        pltpu.async_copy(tab.at[idx_v.at[pl.ds(0, _CHUNK)]], bufs[0], sem)
        for k in range(_NCHUNK):
            buf = bufs[k % 2]
            pltpu.make_async_copy(tab.at[pl.ds(0, _CHUNK)], buf, sem).wait()
            if k + 1 < _NCHUNK:
                pltpu.async_copy(
                    tab.at[idx_v.at[pl.ds((k + 1) * _CHUNK, _CHUNK)]],
                    bufs[(k + 1) % 2], sem)

            def sel(c, _, buf=buf):
                col = jnp.broadcast_to(f * DIM + c, (_L,)).astype(jnp.int32)

                def blk(jb, _):
                    rvec = jb * _L + lane
                    g = plsc.load_gather(buf, [rvec, col])
                    tr_v[c, pl.ds(jb * _L, _L)] = g
                    return 0

                lax.fori_loop(0, _CHUNK // _L, blk, 0)
                return 0

            lax.fori_loop(0, DIM, sel, 0)
            pltpu.sync_copy(tr_v, out.at[:, pl.ds(base + k * _CHUNK, _CHUNK)])


def kernel(values_f1, lengths_f1, values_f2, lengths_f2,
           values_f3, lengths_f3, values_f4, lengths_f4,
           table_f1, table_f2, table_f3, table_f4):
    tab = jnp.concatenate([table_f1, table_f2, table_f3, table_f4], axis=1)
    o1, o2, o3, o4 = _gather4(values_f1, values_f2, values_f3, values_f4, tab)
    return (o1.T, lengths_f1, o2.T, lengths_f2,
            o3.T, lengths_f3, o4.T, lengths_f4)


# speedup vs baseline: 1.0123x; 1.0123x over previous
import functools

import jax
import jax.numpy as jnp
from jax import lax
from jax.experimental import pallas as pl
from jax.experimental.pallas import tpu as pltpu
from jax.experimental.pallas import tpu_sc as plsc

VOCAB = 1000000
DIM = 32
NVALS = 81920
CDIM = 4 * DIM

_info = plsc.get_sparse_core_info()
_NC, _NS = _info.num_cores, _info.num_subcores
_NW = _NC * _NS
_BPW = NVALS // _NW
_CHUNK = 320
_NCHUNK = _BPW // _CHUNK


_mesh = plsc.VectorSubcoreMesh(core_axis_name="c", subcore_axis_name="s")


@functools.partial(
    pl.kernel,
    mesh=_mesh,
    out_type=[jax.ShapeDtypeStruct((NVALS, CDIM), jnp.float32)] * 4,
    scratch_types=[
        pltpu.VMEM((_BPW,), jnp.int32),
        pltpu.VMEM((_CHUNK, CDIM), jnp.float32),
        pltpu.VMEM((_CHUNK, CDIM), jnp.float32),
        pltpu.SemaphoreType.DMA,
    ],
)
def _gather4(v1, v2, v3, v4, tab, o1, o2, o3, o4,
             idx_v, rows_a, rows_b, sem):
    wid = lax.axis_index("s") * _NC + lax.axis_index("c")
    base = wid * _BPW
    bufs = (rows_a, rows_b)
    for vals, out in ((v1, o1), (v2, o2), (v3, o3), (v4, o4)):
        pltpu.sync_copy(vals.at[pl.ds(base, _BPW)], idx_v)
        pltpu.async_copy(tab.at[idx_v.at[pl.ds(0, _CHUNK)]], bufs[0], sem)
        for k in range(_NCHUNK):
            buf = bufs[k % 2]
            pltpu.make_async_copy(tab.at[pl.ds(0, _CHUNK)], buf, sem).wait()
            if k + 1 < _NCHUNK:
                pltpu.async_copy(
                    tab.at[idx_v.at[pl.ds((k + 1) * _CHUNK, _CHUNK)]],
                    bufs[(k + 1) % 2], sem)
            pltpu.sync_copy(buf, out.at[pl.ds(base + k * _CHUNK, _CHUNK)])


def kernel(values_f1, lengths_f1, values_f2, lengths_f2,
           values_f3, lengths_f3, values_f4, lengths_f4,
           table_f1, table_f2, table_f3, table_f4):
    tab = jnp.concatenate([table_f1, table_f2, table_f3, table_f4], axis=1)
    o1, o2, o3, o4 = _gather4(values_f1, values_f2, values_f3, values_f4, tab)
    return (o1[:, 0:DIM], lengths_f1,
            o2[:, DIM:2 * DIM], lengths_f2,
            o3[:, 2 * DIM:3 * DIM], lengths_f3,
            o4[:, 3 * DIM:4 * DIM], lengths_f4)
